# Initial kernel scaffold; baseline (speedup 1.0000x reference)
import jax
import jax.numpy as jnp
from jax.experimental import pallas as pl


def kernel(x, edge_index, batch, W_emb, b_emb, eps, W1, b1, g1, be1, W2, b2, g2, be2):
    raise NotImplementedError("write your pallas kernel here")


# speedup vs baseline: 2.4339x; 2.4339x over previous
import functools

import jax
import jax.numpy as jnp
from jax import lax
from jax.experimental import pallas as pl
from jax.experimental.pallas import tpu as pltpu
from jax.experimental.pallas import tpu_sc as plsc

N = 50000
E = 800000
H = 64
B = 512

NC = 2
NS = 16
NH = N // NC
NP = N // 2
NHP = NH // 2
RPT = 784
NPAD = NS * RPT
EPT = E // NS
CH = 128
NFULL = EPT // CH
TAIL = EPT - NFULL * CH
RPT_LAST = NHP - (NS - 1) * RPT

ROWBLK = 2000
NSTEPS = N // ROWBLK


def _sc_body(h_hbm, src_hbm, dst_hbm, agg_hbm,
             acc, gi_v, pr_v, gi_t, pr_t, rows_v, sem):
  c = lax.axis_index("c")
  s = lax.axis_index("s")
  half_base = c * NH

  zf = jnp.zeros((16,), jnp.float32)
  def _zfill(k, _):
    rows_v[k // 8, pl.ds((k % 8) * 16, 16)] = zf
    return _
  lax.fori_loop(0, CH * 8, _zfill, None)
  zbase = s * RPT
  for piece in range(6):
    pltpu.sync_copy(rows_v, acc.at[pl.ds(zbase + piece * CH, CH)])
  pltpu.sync_copy(rows_v.at[pl.ds(0, 16)], acc.at[pl.ds(zbase + 6 * CH, 16)])
  plsc.subcore_barrier()

  def _chunk(gi_buf, pr_buf, rows_buf, base, k):
    pltpu.sync_copy(src_hbm.at[pl.ds(base, k)], gi_buf)
    pltpu.sync_copy(dst_hbm.at[pl.ds(base, k)], pr_buf)
    for r in range(k // 16):
      sl = pl.ds(r * 16, 16)
      dl = pr_buf[sl] - half_base
      ok = (dl >= 0) & (dl < NH)
      gi_buf[sl] = gi_buf[sl] + (dl & 1) * N
      pr_buf[sl] = jnp.where(ok, dl >> 1, NHP)
    pltpu.async_copy(h_hbm.at[gi_buf], rows_buf, sem).wait()
    pltpu.sync_copy(rows_buf, acc.at[pr_buf], add=True)

  estart = s * EPT
  def _eloop(j, _):
    _chunk(gi_v, pr_v, rows_v, estart + j * CH, CH)
    return _
  lax.fori_loop(0, NFULL, _eloop, None)
  _chunk(gi_t, pr_t, rows_v.at[pl.ds(0, TAIL)], estart + NFULL * CH, TAIL)

  plsc.subcore_barrier()

  pltpu.sync_copy(acc.at[pl.ds(s * RPT, RPT)],
                  agg_hbm.at[c, pl.ds(s * RPT, RPT)])


_sc_scatter = functools.partial(
    pl.kernel,
    out_type=jax.ShapeDtypeStruct((NC, NPAD, 2 * H), jnp.float32),
    mesh=plsc.VectorSubcoreMesh(core_axis_name="c", subcore_axis_name="s",
                                num_cores=NC, num_subcores=NS),
    scratch_types=[
        pltpu.VMEM_SHARED((NPAD, 2 * H), jnp.float32),
        pltpu.VMEM((CH,), jnp.int32),
        pltpu.VMEM((CH,), jnp.int32),
        pltpu.VMEM((TAIL,), jnp.int32),
        pltpu.VMEM((TAIL,), jnp.int32),
        pltpu.VMEM((CH, 2 * H), jnp.float32),
        pltpu.SemaphoreType.DMA,
    ],
)(_sc_body)


def _lohi_store(p, h, out_ref):
  z = jnp.zeros_like(h)
  out_ref[:, 0:H] = jnp.where(p == 0, h, z)
  out_ref[:, H:2 * H] = jnp.where(p == 0, z, h)


def _emb_body(x_ref, w_ref, b_ref, h_ref):
  h = (jax.lax.dot_general(x_ref[...], w_ref[...], (((1,), (0,)), ((), ())),
                           preferred_element_type=jnp.float32)
       + b_ref[...])
  _lohi_store(pl.program_id(0), h, h_ref)


def _emb(xp, wp, b):
  return pl.pallas_call(
      _emb_body,
      grid=(2, NSTEPS),
      in_specs=[
          pl.BlockSpec((ROWBLK, 16), lambda p, i: (i, 0)),
          pl.BlockSpec((16, H), lambda p, i: (0, 0)),
          pl.BlockSpec((1, H), lambda p, i: (0, 0)),
      ],
      out_specs=pl.BlockSpec((ROWBLK, 2 * H), lambda p, i: (p * NSTEPS + i, 0)),
      out_shape=jax.ShapeDtypeStruct((2 * N, 2 * H), jnp.float32),
  )(xp, wp, b)


def _stats_body(scal_ref, h_ref, agg_ref, a_ref, s1_ref, g_ref, s1_acc, g_acc):
  i = pl.program_id(0)
  a = scal_ref[0, 0] * h_ref[:, 0:H] + agg_ref[...]
  a_ref[...] = a

  @pl.when(i == 0)
  def _():
    s1_acc[...] = jnp.zeros_like(s1_acc)
    g_acc[...] = jnp.zeros_like(g_acc)

  s1_acc[...] += jnp.sum(a, axis=0, keepdims=True)
  g_acc[...] += jax.lax.dot_general(a, a, (((0,), (0,)), ((), ())),
                                    preferred_element_type=jnp.float32)

  @pl.when(i == NSTEPS - 1)
  def _():
    s1_ref[...] = s1_acc[...]
    g_ref[...] = g_acc[...]


def _stats(scal, h, agg):
  return pl.pallas_call(
      _stats_body,
      grid=(NSTEPS,),
      in_specs=[
          pl.BlockSpec((1, 1), lambda i: (0, 0)),
          pl.BlockSpec((ROWBLK, 2 * H), lambda i: (i, 0)),
          pl.BlockSpec((ROWBLK, H), lambda i: (i, 0)),
      ],
      out_specs=[
          pl.BlockSpec((ROWBLK, H), lambda i: (i, 0)),
          pl.BlockSpec((1, H), lambda i: (0, 0)),
          pl.BlockSpec((H, H), lambda i: (0, 0)),
      ],
      out_shape=[
          jax.ShapeDtypeStruct((N, H), jnp.float32),
          jax.ShapeDtypeStruct((1, H), jnp.float32),
          jax.ShapeDtypeStruct((H, H), jnp.float32),
      ],
      scratch_shapes=[
          pltpu.VMEM((1, H), jnp.float32),
          pltpu.VMEM((H, H), jnp.float32),
      ],
  )(scal, h, agg)


def _mlp_body(a_ref, s1_ref, g_ref, w1_ref, b1_ref, g1_ref, be1_ref,
              w2_ref, b2_ref, z2_ref, s2_ref, q2_ref, s2_acc, q2_acc):
  i = pl.program_id(0)
  inv_n = 1.0 / N
  w1 = w1_ref[...]
  mu = jax.lax.dot_general(s1_ref[...] * inv_n, w1, (((1,), (0,)), ((), ())),
                           preferred_element_type=jnp.float32)
  gw = jax.lax.dot_general(g_ref[...] * inv_n, w1, (((1,), (0,)), ((), ())),
                           preferred_element_type=jnp.float32)
  var = jnp.sum(w1 * gw, axis=0, keepdims=True) - mu * mu
  sc1 = g1_ref[...] * jax.lax.rsqrt(var + 1e-5)
  sh1 = be1_ref[...] - mu * sc1

  z1 = jax.lax.dot_general(a_ref[...], w1, (((1,), (0,)), ((), ())),
                           preferred_element_type=jnp.float32)
  u = jnp.maximum(z1 * sc1 + sh1, 0.0)
  z2 = (jax.lax.dot_general(u, w2_ref[...], (((1,), (0,)), ((), ())),
                            preferred_element_type=jnp.float32)
        + b2_ref[...])
  z2_ref[...] = z2

  @pl.when(i == 0)
  def _():
    s2_acc[...] = jnp.zeros_like(s2_acc)
    q2_acc[...] = jnp.zeros_like(q2_acc)

  s2_acc[...] += jnp.sum(z2, axis=0, keepdims=True)
  q2_acc[...] += jnp.sum(z2 * z2, axis=0, keepdims=True)

  @pl.when(i == NSTEPS - 1)
  def _():
    s2_ref[...] = s2_acc[...]
    q2_ref[...] = q2_acc[...]


def _mlp(a, s1, g, w1, b1, g1, be1, w2, b2):
  return pl.pallas_call(
      _mlp_body,
      grid=(NSTEPS,),
      in_specs=[
          pl.BlockSpec((ROWBLK, H), lambda i: (i, 0)),
          pl.BlockSpec((1, H), lambda i: (0, 0)),
          pl.BlockSpec((H, H), lambda i: (0, 0)),
          pl.BlockSpec((H, 2 * H), lambda i: (0, 0)),
          pl.BlockSpec((1, 2 * H), lambda i: (0, 0)),
          pl.BlockSpec((1, 2 * H), lambda i: (0, 0)),
          pl.BlockSpec((1, 2 * H), lambda i: (0, 0)),
          pl.BlockSpec((2 * H, H), lambda i: (0, 0)),
          pl.BlockSpec((1, H), lambda i: (0, 0)),
      ],
      out_specs=[
          pl.BlockSpec((ROWBLK, H), lambda i: (i, 0)),
          pl.BlockSpec((1, H), lambda i: (0, 0)),
          pl.BlockSpec((1, H), lambda i: (0, 0)),
      ],
      out_shape=[
          jax.ShapeDtypeStruct((N, H), jnp.float32),
          jax.ShapeDtypeStruct((1, H), jnp.float32),
          jax.ShapeDtypeStruct((1, H), jnp.float32),
      ],
      scratch_shapes=[
          pltpu.VMEM((1, H), jnp.float32),
          pltpu.VMEM((1, H), jnp.float32),
      ],
  )(a, s1, g, w1, b1, g1, be1, w2, b2)


def _norm_body(z2_ref, s2_ref, q2_ref, g2_ref, be2_ref, h_ref):
  inv_n = 1.0 / N
  mean = s2_ref[...] * inv_n
  var = q2_ref[...] * inv_n - mean * mean
  sc = g2_ref[...] * jax.lax.rsqrt(var + 1e-5)
  sh = be2_ref[...] - mean * sc
  h = jnp.maximum(z2_ref[...] * sc + sh, 0.0)
  _lohi_store(pl.program_id(0), h, h_ref)


def _norm(z2, s2, q2, g2, be2):
  return pl.pallas_call(
      _norm_body,
      grid=(2, NSTEPS),
      in_specs=[
          pl.BlockSpec((ROWBLK, H), lambda p, i: (i, 0)),
          pl.BlockSpec((1, H), lambda p, i: (0, 0)),
          pl.BlockSpec((1, H), lambda p, i: (0, 0)),
          pl.BlockSpec((1, H), lambda p, i: (0, 0)),
          pl.BlockSpec((1, H), lambda p, i: (0, 0)),
      ],
      out_specs=pl.BlockSpec((ROWBLK, 2 * H), lambda p, i: (p * NSTEPS + i, 0)),
      out_shape=jax.ShapeDtypeStruct((2 * N, 2 * H), jnp.float32),
  )(z2, s2, q2, g2, be2)


def _pool_body(b_ref, h_ref, out_ref, sum_acc, cnt_acc):
  i = pl.program_id(0)

  @pl.when(i == 0)
  def _():
    sum_acc[...] = jnp.zeros_like(sum_acc)
    cnt_acc[...] = jnp.zeros_like(cnt_acc)

  gid = b_ref[0, 0, :]
  onehot = (gid[:, None] ==
            lax.broadcasted_iota(jnp.int32, (ROWBLK, B), 1)
            ).astype(jnp.float32)
  sum_acc[...] += jax.lax.dot_general(
      onehot, h_ref[:, 0:H], (((0,), (0,)), ((), ())),
      preferred_element_type=jnp.float32)
  cnt_acc[...] += jnp.sum(onehot, axis=0, keepdims=True)

  @pl.when(i == NSTEPS - 1)
  def _():
    cnt = jnp.maximum(cnt_acc[...], 1.0)
    inv = (1.0 / cnt)[0, :]
    out_ref[...] = sum_acc[...] * inv[:, None]


def _pool(batch3, h):
  return pl.pallas_call(
      _pool_body,
      grid=(NSTEPS,),
      in_specs=[
          pl.BlockSpec((1, 1, ROWBLK), lambda i: (i, 0, 0)),
          pl.BlockSpec((ROWBLK, 2 * H), lambda i: (i, 0)),
      ],
      out_specs=pl.BlockSpec((B, H), lambda i: (0, 0)),
      out_shape=jax.ShapeDtypeStruct((B, H), jnp.float32),
      scratch_shapes=[
          pltpu.VMEM((B, H), jnp.float32),
          pltpu.VMEM((1, B), jnp.float32),
      ],
  )(batch3, h)


def kernel(x, edge_index, batch, W_emb, b_emb, eps, W1, b1, g1, be1,
           W2, b2, g2, be2):
  xp = jnp.pad(x, ((0, 0), (0, 16 - x.shape[1])))
  wp = jnp.pad(W_emb, ((0, 16 - W_emb.shape[0]), (0, 0)))
  src = edge_index[0]
  dst = edge_index[1]
  batch3 = batch.reshape(NSTEPS, 1, ROWBLK)

  h = _emb(xp, wp, b_emb.reshape(1, H))
  for i in range(4):
    agg = _sc_scatter(h, src, dst)[:, :NHP, :].reshape(N, H)
    scal = (1.0 + eps[i]).reshape(1, 1)
    a, s1, gmat = _stats(scal, h, agg)
    z2, s2, q2 = _mlp(a, s1, gmat, W1[i], b1[i].reshape(1, 2 * H),
                      g1[i].reshape(1, 2 * H), be1[i].reshape(1, 2 * H),
                      W2[i], b2[i].reshape(1, H))
    h = _norm(z2, s2, q2, g2[i].reshape(1, H), be2[i].reshape(1, H))
  return _pool(batch3, h)
